# baseline (speedup 1.0000x reference)
import math

import jax
import jax.numpy as jnp
from jax.experimental import pallas as pl
from jax.experimental.pallas import tpu as pltpu

_ITERS = 20
_BPP = 2


def _ot_body(x_ref, y_ref, inv_eps_ref, out_ref, k_ref):
    n = x_ref.shape[1]
    inv_n = 1.0 / n
    log_n = math.log(n)
    inv_eps = inv_eps_ref[0, 0]

    u0s = []
    for j in range(_BPP):
        x = x_ref[j]
        y = y_ref[j]

        xy = jax.lax.dot_general(
            x, y, (((1,), (1,)), ((), ())),
            preferred_element_type=jnp.float32,
            precision=jax.lax.Precision.DEFAULT,
        )
        x2 = jnp.sum(x * x, axis=1, keepdims=True)
        yy = y * y
        ones_row = jnp.ones((1, yy.shape[1]), jnp.float32)
        y2row = jax.lax.dot_general(
            ones_row, yy, (((1,), (1,)), ((), ())),
            preferred_element_type=jnp.float32,
            precision=jax.lax.Precision.HIGHEST,
        )
        c_mat = jnp.maximum(x2 + y2row - 2.0 * xy, 0.0)

        r = jnp.min(c_mat, axis=1, keepdims=True)
        c = jnp.min(c_mat - r, axis=0, keepdims=True)
        rbar = jnp.mean(r)
        t1 = (c + rbar - c_mat) * inv_eps
        m1 = jnp.max(t1, axis=0, keepdims=True)
        s1 = jnp.sum(jnp.exp(t1 - m1), axis=0, keepdims=True)
        lv1 = (-log_n) - m1 - jnp.log(s1)
        t2 = t1 + ((r - rbar) * inv_eps + lv1)
        m2 = jnp.max(t2, axis=1, keepdims=True)
        e2 = jnp.exp(t2 - m2)
        s2 = jnp.sum(e2, axis=1, keepdims=True)
        k_ref[j] = e2
        u0s.append(inv_n / s2)

    ch = 32

    def colsum_ku(j, u):
        acc_a = acc_b = None
        for i0 in range(0, n, ch):
            kc = k_ref[j, i0:i0 + ch, :] * u[i0:i0 + ch, :]
            p = jnp.sum(kc, axis=0, keepdims=True)
            if i0 % (2 * ch) == 0:
                acc_a = p if acc_a is None else acc_a + p
            else:
                acc_b = p if acc_b is None else acc_b + p
        return acc_a + acc_b

    def rowsum_kv(j, v):
        parts = []
        for i0 in range(0, n, ch):
            kc = k_ref[j, i0:i0 + ch, :] * v
            parts.append(jnp.sum(kc, axis=1, keepdims=True))
        return jnp.concatenate(parts, axis=0)

    def body(_, us):
        outs = []
        for j in range(_BPP):
            v = inv_n / colsum_ku(j, us[j])
            outs.append(inv_n / rowsum_kv(j, v))
        return tuple(outs)

    us = jax.lax.fori_loop(0, _ITERS - 1, body, tuple(u0s))

    for j in range(_BPP):
        v = inv_n / colsum_ku(j, us[j])
        kv = k_ref[j] * v
        den = rowsum_kv(j, v)
        num = jax.lax.dot_general(
            kv, y_ref[j], (((1,), (0,)), ((), ())),
            preferred_element_type=jnp.float32,
            precision=jax.lax.Precision.DEFAULT,
        )
        out_ref[j] = num / den


def kernel(x, y, eps):
    b, n, d = x.shape
    inv_eps = (1.0 / eps).reshape(1, 1).astype(jnp.float32)
    return pl.pallas_call(
        _ot_body,
        grid=(b // _BPP,),
        in_specs=[
            pl.BlockSpec((_BPP, n, d), lambda i: (i, 0, 0)),
            pl.BlockSpec((_BPP, n, d), lambda i: (i, 0, 0)),
            pl.BlockSpec((1, 1), lambda i: (0, 0), memory_space=pltpu.SMEM),
        ],
        out_specs=pl.BlockSpec((_BPP, n, d), lambda i: (i, 0, 0)),
        out_shape=jax.ShapeDtypeStruct((b, n, d), jnp.float32),
        scratch_shapes=[pltpu.VMEM((_BPP, n, n), jnp.float32)],
        compiler_params=pltpu.CompilerParams(
            dimension_semantics=("arbitrary",),
        ),
    )(x, y, inv_eps)


# speedup vs baseline: 1.1194x; 1.1194x over previous
import math

import jax
import jax.numpy as jnp
from jax.experimental import pallas as pl
from jax.experimental.pallas import tpu as pltpu

_ITERS = 20
_BPP = 2


def _ot_body(x_ref, y_ref, inv_eps_ref, out_ref, k_ref):
    n = x_ref.shape[1]
    inv_n = 1.0 / n
    log_n = math.log(n)
    inv_eps = inv_eps_ref[0, 0]

    u0s = []
    for j in range(_BPP):
        x = x_ref[j]
        y = y_ref[j]

        xy = jax.lax.dot_general(
            x, y, (((1,), (1,)), ((), ())),
            preferred_element_type=jnp.float32,
            precision=jax.lax.Precision.DEFAULT,
        )
        x2 = jnp.sum(x * x, axis=1, keepdims=True)
        yy = y * y
        ones_row = jnp.ones((1, yy.shape[1]), jnp.float32)
        y2row = jax.lax.dot_general(
            ones_row, yy, (((1,), (1,)), ((), ())),
            preferred_element_type=jnp.float32,
            precision=jax.lax.Precision.HIGHEST,
        )
        c_mat = jnp.maximum(x2 + y2row - 2.0 * xy, 0.0)
        a0 = c_mat * (-inv_eps)

        m1 = jnp.max(a0, axis=0, keepdims=True)
        s1 = jnp.sum(jnp.exp(a0 - m1), axis=0, keepdims=True)
        lv1 = (-log_n) - m1 - jnp.log(s1)
        t2 = a0 + lv1
        m2 = jnp.max(t2, axis=1, keepdims=True)
        e2 = jnp.exp(t2 - m2)
        s2 = jnp.sum(e2, axis=1, keepdims=True)
        k_ref[j] = e2
        u0s.append(inv_n / s2)

    def body(_, us):
        outs = []
        for j in range(_BPP):
            s = jnp.sum(k_ref[j] * us[j], axis=0, keepdims=True)
            v = inv_n / s
            z = jnp.sum(k_ref[j] * v, axis=1, keepdims=True)
            outs.append(inv_n / z)
        return tuple(outs)

    us = jax.lax.fori_loop(0, _ITERS - 1, body, tuple(u0s))

    for j in range(_BPP):
        s = jnp.sum(k_ref[j] * us[j], axis=0, keepdims=True)
        v = inv_n / s
        kv = k_ref[j] * v
        den = jnp.sum(kv, axis=1, keepdims=True)
        num = jax.lax.dot_general(
            kv, y_ref[j], (((1,), (0,)), ((), ())),
            preferred_element_type=jnp.float32,
            precision=jax.lax.Precision.DEFAULT,
        )
        out_ref[j] = num / den


def kernel(x, y, eps):
    b, n, d = x.shape
    inv_eps = (1.0 / eps).reshape(1, 1).astype(jnp.float32)
    return pl.pallas_call(
        _ot_body,
        grid=(b // _BPP,),
        in_specs=[
            pl.BlockSpec((_BPP, n, d), lambda i: (i, 0, 0)),
            pl.BlockSpec((_BPP, n, d), lambda i: (i, 0, 0)),
            pl.BlockSpec((1, 1), lambda i: (0, 0), memory_space=pltpu.SMEM),
        ],
        out_specs=pl.BlockSpec((_BPP, n, d), lambda i: (i, 0, 0)),
        out_shape=jax.ShapeDtypeStruct((b, n, d), jnp.float32),
        scratch_shapes=[pltpu.VMEM((_BPP, n, n), jnp.float32)],
        compiler_params=pltpu.CompilerParams(
            dimension_semantics=("arbitrary",),
        ),
    )(x, y, inv_eps)
